# Initial kernel scaffold; baseline (speedup 1.0000x reference)
import jax
import jax.numpy as jnp
from jax.experimental import pallas as pl


def kernel(pts_nd, cc):
    raise NotImplementedError("write your pallas kernel here")


# speedup vs baseline: 24.4916x; 24.4916x over previous
import functools

import jax
import jax.numpy as jnp
from jax.experimental import pallas as pl

_NN = 10
_SIGMA = 5.0
_BIG = 3.0e38


def _nnencode_block(pts_ref, cc_ref, out_ref):
    pts = pts_ref[...]
    x = pts[:, 0:1]
    y = pts[:, 1:2]
    cx = cc_ref[0:1, :]
    cy = cc_ref[1:2, :]
    p2 = x * x + y * y
    c2 = cx * cx + cy * cy
    cross = x * cx + y * cy
    d2 = jnp.maximum(p2 + c2 - 2.0 * cross, 0.0)

    cur = d2
    thr = None
    for _ in range(_NN):
        thr = jnp.min(cur, axis=1, keepdims=True)
        cur = jnp.where(cur <= thr, _BIG, cur)

    keep = d2 <= thr
    w = jnp.where(keep, jnp.exp(d2 * (-1.0 / (2.0 * _SIGMA ** 2))), 0.0)
    s = jnp.sum(w, axis=1, keepdims=True)
    out_ref[...] = jnp.where(keep, w / s, 0.0)


@functools.partial(jax.jit, static_argnames=("block_rows", "interpret"))
def _nnencode(pts_nd, cc, block_rows=512, interpret=False):
    B, S, D = pts_nd.shape
    K = cc.shape[0]
    P = B * S
    pts_flt = pts_nd.reshape(P, D)
    cc_t = cc.T
    grid = (P // block_rows,)
    out = pl.pallas_call(
        _nnencode_block,
        grid=grid,
        in_specs=[
            pl.BlockSpec((block_rows, D), lambda i: (i, 0)),
            pl.BlockSpec((D, K), lambda i: (0, 0)),
        ],
        out_specs=pl.BlockSpec((block_rows, K), lambda i: (i, 0)),
        out_shape=jax.ShapeDtypeStruct((P, K), jnp.float32),
        interpret=interpret,
    )(pts_flt, cc_t)
    return out.reshape(B, S, K)


def kernel(pts_nd, cc):
    return _nnencode(pts_nd, cc)
